# Initial kernel scaffold; baseline (speedup 1.0000x reference)
import jax
import jax.numpy as jnp
from jax.experimental import pallas as pl


def kernel(x, edge_index, W1, a_src1, a_dst1, b1, W2, a_src2, a_dst2, b2, Wlin, blin):
    raise NotImplementedError("write your pallas kernel here")


# speedup vs baseline: 29.6603x; 29.6603x over previous
import dataclasses
import functools

import jax
import jax.numpy as jnp
from jax import lax
from jax.experimental import pallas as pl
from jax.experimental.pallas import tpu as pltpu
from jax.experimental.pallas import tpu_sc as plsc

F32 = jnp.float32

_N = 10000
_E = 320000
_H = 32
_NCORE = 2
_NSUB = 16
_NTILE = _NCORE * _NSUB
_G = 128
_GROUPS_PER_TILE = 80
_E_PAD = _NTILE * _GROUPS_PER_TILE * _G
_N_SH = 10240
_ROWS_T = _N_SH // _NSUB
_ESD_PAD = 20480


def _leaky(a):
    return jnp.where(a >= 0, a, a * jnp.float32(0.2))


def _pre_body(x_ref, w_ref, a2_ref, h_ref, esd_ref):
    h = jnp.dot(x_ref[...], w_ref[...], preferred_element_type=F32,
                precision=lax.Precision.HIGHEST)
    h_ref[...] = h
    esd_ref[...] = jnp.dot(h, a2_ref[...], preferred_element_type=F32,
                           precision=lax.Precision.HIGHEST)


def _node_mm(x, W, a_src, a_dst, blk=400):
    n, f = x.shape
    h_dim = W.shape[1]
    a2 = jnp.stack([a_src, a_dst], axis=1)
    return pl.pallas_call(
        _pre_body,
        grid=(n // blk,),
        in_specs=[
            pl.BlockSpec((blk, f), lambda i: (i, 0)),
            pl.BlockSpec((f, h_dim), lambda i: (0, 0)),
            pl.BlockSpec((h_dim, 2), lambda i: (0, 0)),
        ],
        out_specs=[
            pl.BlockSpec((blk, h_dim), lambda i: (i, 0)),
            pl.BlockSpec((blk, 2), lambda i: (i, 0)),
        ],
        out_shape=[
            jax.ShapeDtypeStruct((n, h_dim), F32),
            jax.ShapeDtypeStruct((n, 2), F32),
        ],
    )(x, W, a2)


def _combine_block(num_ref, dent_ref, h_ref, esd_ref, b_ref):
    nsum = num_ref[0] + num_ref[1]
    dsum = dent_ref[:, 0:1] + dent_ref[:, 1:2]
    a_self = esd_ref[:, 0:1] + esd_ref[:, 1:2]
    p_self = jnp.exp(_leaky(a_self))
    out = (nsum + p_self * h_ref[...]) / (dsum + p_self)
    return out + b_ref[...]


def _mid_body(num_ref, dent_ref, h_ref, esd_ref, b_ref, w_ref, a2_ref,
              h2_ref, esd2_ref):
    x2 = jnp.tanh(_combine_block(num_ref, dent_ref, h_ref, esd_ref, b_ref))
    h2 = jnp.dot(x2, w_ref[...], preferred_element_type=F32,
                 precision=lax.Precision.HIGHEST)
    h2_ref[...] = h2
    esd2_ref[...] = jnp.dot(h2, a2_ref[...], preferred_element_type=F32,
                            precision=lax.Precision.HIGHEST)


def _mid(num, den_t, h, esd, b, W2, a_src2, a_dst2, blk=400):
    n, h_dim = h.shape
    a2 = jnp.stack([a_src2, a_dst2], axis=1)
    return pl.pallas_call(
        _mid_body,
        grid=(n // blk,),
        in_specs=[
            pl.BlockSpec((2, blk, h_dim), lambda i: (0, i, 0)),
            pl.BlockSpec((blk, 2), lambda i: (i, 0)),
            pl.BlockSpec((blk, h_dim), lambda i: (i, 0)),
            pl.BlockSpec((blk, 2), lambda i: (i, 0)),
            pl.BlockSpec((1, h_dim), lambda i: (0, 0)),
            pl.BlockSpec((h_dim, h_dim), lambda i: (0, 0)),
            pl.BlockSpec((h_dim, 2), lambda i: (0, 0)),
        ],
        out_specs=[
            pl.BlockSpec((blk, h_dim), lambda i: (i, 0)),
            pl.BlockSpec((blk, 2), lambda i: (i, 0)),
        ],
        out_shape=[
            jax.ShapeDtypeStruct((n, h_dim), F32),
            jax.ShapeDtypeStruct((n, 2), F32),
        ],
    )(num, den_t, h, esd, b.reshape(1, h_dim), W2, a2)


def _post_body(num_ref, dent_ref, h_ref, esd_ref, b_ref, wlin_ref, blin_ref,
               out_ref):
    xx = _combine_block(num_ref, dent_ref, h_ref, esd_ref, b_ref)
    out_ref[...] = jnp.dot(xx, wlin_ref[...], preferred_element_type=F32,
                           precision=lax.Precision.HIGHEST) + blin_ref[...]


def _post(num, den_t, h, esd, b, Wlin, blin, blk=400):
    n, h_dim = h.shape
    t = Wlin.shape[1]
    return pl.pallas_call(
        _post_body,
        grid=(n // blk,),
        in_specs=[
            pl.BlockSpec((2, blk, h_dim), lambda i: (0, i, 0)),
            pl.BlockSpec((blk, 2), lambda i: (i, 0)),
            pl.BlockSpec((blk, h_dim), lambda i: (i, 0)),
            pl.BlockSpec((blk, 2), lambda i: (i, 0)),
            pl.BlockSpec((1, h_dim), lambda i: (0, 0)),
            pl.BlockSpec((h_dim, t), lambda i: (0, 0)),
            pl.BlockSpec((1, t), lambda i: (0, 0)),
        ],
        out_specs=pl.BlockSpec((blk, t), lambda i: (i, 0)),
        out_shape=jax.ShapeDtypeStruct((n, t), F32),
    )(num, den_t, h, esd, b.reshape(1, h_dim), Wlin, blin.reshape(1, t))


def _edge_body(h_hbm, esd_hbm, src_hbm, dst_hbm, num_hbm, den_hbm,
               esd_loc, srcb, dstb, pbuf, rows, zbuf, num_sh, den_sh):
    core = lax.axis_index("c")
    sub = lax.axis_index("s")
    wid = core * _NSUB + sub

    @pl.loop(0, _G)
    def _zero_rows(r):
        rows[r, pl.ds(0, 16)] = jnp.zeros((16,), F32)
        rows[r, pl.ds(16, 16)] = jnp.zeros((16,), F32)

    @pl.loop(0, _ROWS_T, step=16)
    def _zero_zbuf(i):
        zbuf[pl.ds(i, 16)] = jnp.zeros((16,), F32)

    base = sub * _ROWS_T
    for i in range(_ROWS_T // _G):
        pltpu.sync_copy(rows, num_sh.at[pl.ds(base + i * _G, _G)])
    pltpu.sync_copy(zbuf, den_sh.at[pl.ds(base, _ROWS_T)])

    pltpu.sync_copy(esd_hbm, esd_loc)

    plsc.subcore_barrier()

    @pl.loop(0, _GROUPS_PER_TILE)
    def _group(g):
        crow = wid * _GROUPS_PER_TILE + g
        pltpu.sync_copy(src_hbm.at[pl.ds(crow, 1)], srcb)
        pltpu.sync_copy(dst_hbm.at[pl.ds(crow, 1)], dstb)

        @pl.loop(0, _G, step=16)
        def _logits(i):
            sv = srcb[0, pl.ds(i, 16)]
            dv = dstb[0, pl.ds(i, 16)]
            ea = plsc.load_gather(esd_loc, [sv * 2])
            eb = plsc.load_gather(esd_loc, [dv * 2 + 1])
            pbuf[0, pl.ds(i, 16)] = jnp.exp(_leaky(ea + eb))

        pltpu.sync_copy(h_hbm.at[srcb.at[0]], rows)

        @pl.loop(0, _G, step=16)
        def _scale(r0):
            pv = pbuf[0, pl.ds(r0, 16)]
            for u in range(16):
                ps = pv[u]
                rows[r0 + u, pl.ds(0, 16)] = rows[r0 + u, pl.ds(0, 16)] * ps
                rows[r0 + u, pl.ds(16, 16)] = rows[r0 + u, pl.ds(16, 16)] * ps

        pltpu.sync_copy(rows, num_sh.at[dstb.at[0]], add=True)
        pltpu.sync_copy(pbuf.at[0], den_sh.at[dstb.at[0]], add=True)

    plsc.subcore_barrier()

    hbase = core * _N + base

    @pl.when(sub < _NSUB - 1)
    def _():
        pltpu.sync_copy(num_sh.at[pl.ds(base, _ROWS_T)],
                        num_hbm.at[pl.ds(hbase, _ROWS_T)])
        pltpu.sync_copy(den_sh.at[pl.ds(base, _ROWS_T)],
                        den_hbm.at[pl.ds(hbase, _ROWS_T)])

    @pl.when(sub == _NSUB - 1)
    def _():
        last = _N - (_NSUB - 1) * _ROWS_T
        pltpu.sync_copy(num_sh.at[pl.ds(base, last)],
                        num_hbm.at[pl.ds(hbase, last)])
        pltpu.sync_copy(den_sh.at[pl.ds(base, last)],
                        den_hbm.at[pl.ds(hbase, last)])


def _sc_compiler_params():
    cp = pltpu.CompilerParams()
    fields = pltpu.CompilerParams.__dataclass_fields__
    if "needs_layout_passes" in fields:
        cp = dataclasses.replace(cp, needs_layout_passes=False)
    if "use_tc_tiling_on_sc" in fields:
        cp = dataclasses.replace(cp, use_tc_tiling_on_sc=False)
    return cp


def _edge_pass(h, esd_flat, src2d, dst2d):
    mesh = plsc.VectorSubcoreMesh(core_axis_name="c", subcore_axis_name="s")
    fn = pl.kernel(
        _edge_body,
        mesh=mesh,
        compiler_params=_sc_compiler_params(),
        out_type=[
            jax.ShapeDtypeStruct((_NCORE * _N, _H), F32),
            jax.ShapeDtypeStruct((_NCORE * _N,), F32),
        ],
        scratch_types=[
            pltpu.VMEM((_ESD_PAD,), F32),
            pltpu.VMEM((1, _G), jnp.int32),
            pltpu.VMEM((1, _G), jnp.int32),
            pltpu.VMEM((1, _G), F32),
            pltpu.VMEM((_G, _H), F32),
            pltpu.VMEM((_ROWS_T,), F32),
            pltpu.VMEM_SHARED((_N_SH, _H), F32),
            pltpu.VMEM_SHARED((_N_SH,), F32),
        ],
    )
    return fn(h, esd_flat, src2d, dst2d)


def kernel(x, edge_index, W1, a_src1, a_dst1, b1, W2, a_src2, a_dst2, b2,
           Wlin, blin):
    src = edge_index[0]
    dst = edge_index[1]
    pad = _E_PAD - _E
    src2d = jnp.concatenate(
        [src, jnp.zeros((pad,), jnp.int32)]).reshape(_E_PAD // _G, _G)
    dst2d = jnp.concatenate(
        [dst, jnp.full((pad,), _N, jnp.int32)]).reshape(_E_PAD // _G, _G)

    h1, esd1 = _node_mm(x, W1, a_src1, a_dst1)
    esd1_flat = jnp.concatenate(
        [esd1.reshape(-1), jnp.zeros((_ESD_PAD - 2 * _N,), F32)])
    num1, den1 = _edge_pass(h1, esd1_flat, src2d, dst2d)
    num1 = num1.reshape(_NCORE, _N, _H)
    den1_t = den1.reshape(_NCORE, _N).T
    h2, esd2 = _mid(num1, den1_t, h1, esd1, b1, W2, a_src2, a_dst2)

    esd2_flat = jnp.concatenate(
        [esd2.reshape(-1), jnp.zeros((_ESD_PAD - 2 * _N,), F32)])
    num2, den2 = _edge_pass(h2, esd2_flat, src2d, dst2d)
    num2 = num2.reshape(_NCORE, _N, _H)
    den2_t = den2.reshape(_NCORE, _N).T
    return _post(num2, den2_t, h2, esd2, b2, Wlin, blin)


# speedup vs baseline: 42.3058x; 1.4263x over previous
import dataclasses
import functools

import jax
import jax.numpy as jnp
from jax import lax
from jax.experimental import pallas as pl
from jax.experimental.pallas import tpu as pltpu
from jax.experimental.pallas import tpu_sc as plsc

F32 = jnp.float32

_N = 10000
_E = 320000
_H = 32
_NCORE = 2
_NSUB = 16
_NTILE = _NCORE * _NSUB
_G = 128
_GROUPS_PER_TILE = 80
_E_PAD = _NTILE * _GROUPS_PER_TILE * _G
_N_SH = 10240
_ROWS_T = _N_SH // _NSUB
_ESD_PAD = 20480


def _leaky(a):
    return jnp.where(a >= 0, a, a * jnp.float32(0.2))


def _pre_body(x_ref, w_ref, a2_ref, h_ref, esd_ref):
    h = jnp.dot(x_ref[...], w_ref[...], preferred_element_type=F32,
                precision=lax.Precision.HIGHEST)
    h_ref[...] = h
    esd_ref[...] = jnp.dot(h, a2_ref[...], preferred_element_type=F32,
                           precision=lax.Precision.HIGHEST)


def _node_mm(x, W, a_src, a_dst, blk=400):
    n, f = x.shape
    h_dim = W.shape[1]
    a2 = jnp.stack([a_src, a_dst], axis=1)
    return pl.pallas_call(
        _pre_body,
        grid=(n // blk,),
        in_specs=[
            pl.BlockSpec((blk, f), lambda i: (i, 0)),
            pl.BlockSpec((f, h_dim), lambda i: (0, 0)),
            pl.BlockSpec((h_dim, 2), lambda i: (0, 0)),
        ],
        out_specs=[
            pl.BlockSpec((blk, h_dim), lambda i: (i, 0)),
            pl.BlockSpec((blk, 2), lambda i: (i, 0)),
        ],
        out_shape=[
            jax.ShapeDtypeStruct((n, h_dim), F32),
            jax.ShapeDtypeStruct((n, 2), F32),
        ],
    )(x, W, a2)


def _combine_block(num_ref, dent_ref, h_ref, esd_ref, b_ref):
    nsum = num_ref[0] + num_ref[1]
    dsum = dent_ref[:, 0:1] + dent_ref[:, 1:2]
    a_self = esd_ref[:, 0:1] + esd_ref[:, 1:2]
    p_self = jnp.exp(_leaky(a_self))
    out = (nsum + p_self * h_ref[...]) / (dsum + p_self)
    return out + b_ref[...]


def _mid_body(num_ref, dent_ref, h_ref, esd_ref, b_ref, w_ref, a2_ref,
              h2_ref, esd2_ref):
    x2 = jnp.tanh(_combine_block(num_ref, dent_ref, h_ref, esd_ref, b_ref))
    h2 = jnp.dot(x2, w_ref[...], preferred_element_type=F32,
                 precision=lax.Precision.HIGHEST)
    h2_ref[...] = h2
    esd2_ref[...] = jnp.dot(h2, a2_ref[...], preferred_element_type=F32,
                            precision=lax.Precision.HIGHEST)


def _mid(num, den_t, h, esd, b, W2, a_src2, a_dst2, blk=400):
    n, h_dim = h.shape
    a2 = jnp.stack([a_src2, a_dst2], axis=1)
    return pl.pallas_call(
        _mid_body,
        grid=(n // blk,),
        in_specs=[
            pl.BlockSpec((2, blk, h_dim), lambda i: (0, i, 0)),
            pl.BlockSpec((blk, 2), lambda i: (i, 0)),
            pl.BlockSpec((blk, h_dim), lambda i: (i, 0)),
            pl.BlockSpec((blk, 2), lambda i: (i, 0)),
            pl.BlockSpec((1, h_dim), lambda i: (0, 0)),
            pl.BlockSpec((h_dim, h_dim), lambda i: (0, 0)),
            pl.BlockSpec((h_dim, 2), lambda i: (0, 0)),
        ],
        out_specs=[
            pl.BlockSpec((blk, h_dim), lambda i: (i, 0)),
            pl.BlockSpec((blk, 2), lambda i: (i, 0)),
        ],
        out_shape=[
            jax.ShapeDtypeStruct((n, h_dim), F32),
            jax.ShapeDtypeStruct((n, 2), F32),
        ],
    )(num, den_t, h, esd, b.reshape(1, h_dim), W2, a2)


def _post_body(num_ref, dent_ref, h_ref, esd_ref, b_ref, wlin_ref, blin_ref,
               out_ref):
    xx = _combine_block(num_ref, dent_ref, h_ref, esd_ref, b_ref)
    out_ref[...] = jnp.dot(xx, wlin_ref[...], preferred_element_type=F32,
                           precision=lax.Precision.HIGHEST) + blin_ref[...]


def _post(num, den_t, h, esd, b, Wlin, blin, blk=400):
    n, h_dim = h.shape
    t = Wlin.shape[1]
    return pl.pallas_call(
        _post_body,
        grid=(n // blk,),
        in_specs=[
            pl.BlockSpec((2, blk, h_dim), lambda i: (0, i, 0)),
            pl.BlockSpec((blk, 2), lambda i: (i, 0)),
            pl.BlockSpec((blk, h_dim), lambda i: (i, 0)),
            pl.BlockSpec((blk, 2), lambda i: (i, 0)),
            pl.BlockSpec((1, h_dim), lambda i: (0, 0)),
            pl.BlockSpec((h_dim, t), lambda i: (0, 0)),
            pl.BlockSpec((1, t), lambda i: (0, 0)),
        ],
        out_specs=pl.BlockSpec((blk, t), lambda i: (i, 0)),
        out_shape=jax.ShapeDtypeStruct((n, t), F32),
    )(num, den_t, h, esd, b.reshape(1, h_dim), Wlin, blin.reshape(1, t))


def _edge_body(h_hbm, esd_hbm, src_hbm, dst_hbm, num_hbm, den_hbm,
               esd_loc,
               srcb0, srcb1, dstb0, dstb1, pbuf, rows0, rows1, zbuf,
               num_sh, den_sh,
               sem_idx0, sem_idx1, sem_rows0, sem_rows1):
    core = lax.axis_index("c")
    sub = lax.axis_index("s")
    wid = core * _NSUB + sub
    srcb = (srcb0, srcb1)
    dstb = (dstb0, dstb1)
    rows = (rows0, rows1)
    sem_idx = (sem_idx0, sem_idx1)
    sem_rows = (sem_rows0, sem_rows1)

    @pl.loop(0, _G)
    def _zero_rows(r):
        rows0[r, pl.ds(0, 16)] = jnp.zeros((16,), F32)
        rows0[r, pl.ds(16, 16)] = jnp.zeros((16,), F32)

    @pl.loop(0, _ROWS_T, step=16)
    def _zero_zbuf(i):
        zbuf[pl.ds(i, 16)] = jnp.zeros((16,), F32)

    base = sub * _ROWS_T
    for i in range(_ROWS_T // _G):
        pltpu.sync_copy(rows0, num_sh.at[pl.ds(base + i * _G, _G)])
    pltpu.sync_copy(zbuf, den_sh.at[pl.ds(base, _ROWS_T)])

    pltpu.sync_copy(esd_hbm, esd_loc)

    plsc.subcore_barrier()

    gbase = wid * _GROUPS_PER_TILE

    pltpu.sync_copy(src_hbm.at[pl.ds(gbase, 1)], srcb0)
    pltpu.sync_copy(dst_hbm.at[pl.ds(gbase, 1)], dstb0)
    pltpu.async_copy(h_hbm.at[srcb0.at[0]], rows0, sem_rows0)
    pltpu.async_copy(src_hbm.at[pl.ds(gbase + 1, 1)], srcb1, sem_idx1)
    pltpu.async_copy(dst_hbm.at[pl.ds(gbase + 1, 1)], dstb1, sem_idx1)

    def _body(g, b):
        o = 1 - b
        pltpu.make_async_copy(h_hbm.at[srcb[b].at[0]], rows[b],
                              sem_rows[b]).wait()

        @pl.when(g < _GROUPS_PER_TILE - 1)
        def _():
            pltpu.make_async_copy(src_hbm.at[pl.ds(gbase + g + 1, 1)],
                                  srcb[o], sem_idx[o]).wait()
            pltpu.make_async_copy(dst_hbm.at[pl.ds(gbase + g + 1, 1)],
                                  dstb[o], sem_idx[o]).wait()
            pltpu.async_copy(h_hbm.at[srcb[o].at[0]], rows[o], sem_rows[o])

        @pl.loop(0, _G, step=16)
        def _logits(i):
            sv = srcb[b][0, pl.ds(i, 16)]
            dv = dstb[b][0, pl.ds(i, 16)]
            ea = plsc.load_gather(esd_loc, [sv * 2])
            eb = plsc.load_gather(esd_loc, [dv * 2 + 1])
            pbuf[0, pl.ds(i, 16)] = jnp.exp(_leaky(ea + eb))

        @pl.loop(0, _G, step=16)
        def _scale(r0):
            pv = pbuf[0, pl.ds(r0, 16)]
            for u in range(16):
                ps = pv[u]
                rows[b][r0 + u, pl.ds(0, 16)] = \
                    rows[b][r0 + u, pl.ds(0, 16)] * ps
                rows[b][r0 + u, pl.ds(16, 16)] = \
                    rows[b][r0 + u, pl.ds(16, 16)] * ps

        pltpu.sync_copy(rows[b], num_sh.at[dstb[b].at[0]], add=True)
        pltpu.sync_copy(pbuf.at[0], den_sh.at[dstb[b].at[0]], add=True)

        @pl.when(g < _GROUPS_PER_TILE - 2)
        def _():
            pltpu.async_copy(src_hbm.at[pl.ds(gbase + g + 2, 1)], srcb[b],
                             sem_idx[b])
            pltpu.async_copy(dst_hbm.at[pl.ds(gbase + g + 2, 1)], dstb[b],
                             sem_idx[b])

    @pl.loop(0, _GROUPS_PER_TILE, step=2)
    def _group(g):
        _body(g, 0)
        _body(g + 1, 1)

    plsc.subcore_barrier()

    hbase = core * _N + base

    @pl.when(sub < _NSUB - 1)
    def _():
        pltpu.sync_copy(num_sh.at[pl.ds(base, _ROWS_T)],
                        num_hbm.at[pl.ds(hbase, _ROWS_T)])
        pltpu.sync_copy(den_sh.at[pl.ds(base, _ROWS_T)],
                        den_hbm.at[pl.ds(hbase, _ROWS_T)])

    @pl.when(sub == _NSUB - 1)
    def _():
        last = _N - (_NSUB - 1) * _ROWS_T
        pltpu.sync_copy(num_sh.at[pl.ds(base, last)],
                        num_hbm.at[pl.ds(hbase, last)])
        pltpu.sync_copy(den_sh.at[pl.ds(base, last)],
                        den_hbm.at[pl.ds(hbase, last)])


def _sc_compiler_params():
    cp = pltpu.CompilerParams()
    fields = pltpu.CompilerParams.__dataclass_fields__
    if "needs_layout_passes" in fields:
        cp = dataclasses.replace(cp, needs_layout_passes=False)
    if "use_tc_tiling_on_sc" in fields:
        cp = dataclasses.replace(cp, use_tc_tiling_on_sc=False)
    return cp


def _edge_pass(h, esd_flat, src2d, dst2d):
    mesh = plsc.VectorSubcoreMesh(core_axis_name="c", subcore_axis_name="s")
    fn = pl.kernel(
        _edge_body,
        mesh=mesh,
        compiler_params=_sc_compiler_params(),
        out_type=[
            jax.ShapeDtypeStruct((_NCORE * _N, _H), F32),
            jax.ShapeDtypeStruct((_NCORE * _N,), F32),
        ],
        scratch_types=[
            pltpu.VMEM((_ESD_PAD,), F32),
            pltpu.VMEM((1, _G), jnp.int32),
            pltpu.VMEM((1, _G), jnp.int32),
            pltpu.VMEM((1, _G), jnp.int32),
            pltpu.VMEM((1, _G), jnp.int32),
            pltpu.VMEM((1, _G), F32),
            pltpu.VMEM((_G, _H), F32),
            pltpu.VMEM((_G, _H), F32),
            pltpu.VMEM((_ROWS_T,), F32),
            pltpu.VMEM_SHARED((_N_SH, _H), F32),
            pltpu.VMEM_SHARED((_N_SH,), F32),
            pltpu.SemaphoreType.DMA,
            pltpu.SemaphoreType.DMA,
            pltpu.SemaphoreType.DMA,
            pltpu.SemaphoreType.DMA,
        ],
    )
    return fn(h, esd_flat, src2d, dst2d)


def kernel(x, edge_index, W1, a_src1, a_dst1, b1, W2, a_src2, a_dst2, b2,
           Wlin, blin):
    src = edge_index[0]
    dst = edge_index[1]
    pad = _E_PAD - _E
    src2d = jnp.concatenate(
        [src, jnp.zeros((pad,), jnp.int32)]).reshape(_E_PAD // _G, _G)
    dst2d = jnp.concatenate(
        [dst, jnp.full((pad,), _N, jnp.int32)]).reshape(_E_PAD // _G, _G)

    h1, esd1 = _node_mm(x, W1, a_src1, a_dst1)
    esd1_flat = jnp.concatenate(
        [esd1.reshape(-1), jnp.zeros((_ESD_PAD - 2 * _N,), F32)])
    num1, den1 = _edge_pass(h1, esd1_flat, src2d, dst2d)
    num1 = num1.reshape(_NCORE, _N, _H)
    den1_t = den1.reshape(_NCORE, _N).T
    h2, esd2 = _mid(num1, den1_t, h1, esd1, b1, W2, a_src2, a_dst2)

    esd2_flat = jnp.concatenate(
        [esd2.reshape(-1), jnp.zeros((_ESD_PAD - 2 * _N,), F32)])
    num2, den2 = _edge_pass(h2, esd2_flat, src2d, dst2d)
    num2 = num2.reshape(_NCORE, _N, _H)
    den2_t = den2.reshape(_NCORE, _N).T
    return _post(num2, den2_t, h2, esd2, b2, Wlin, blin)


# speedup vs baseline: 48.3921x; 1.1439x over previous
import dataclasses
import functools

import jax
import jax.numpy as jnp
from jax import lax
from jax.experimental import pallas as pl
from jax.experimental.pallas import tpu as pltpu
from jax.experimental.pallas import tpu_sc as plsc

F32 = jnp.float32

_N = 10000
_E = 320000
_H = 32
_NCORE = 2
_NSUB = 16
_NTILE = _NCORE * _NSUB
_G = 128
_GROUPS_PER_TILE = 80
_E_PAD = _NTILE * _GROUPS_PER_TILE * _G
_N_SH = 10240
_ROWS_T = _N_SH // _NSUB
_ESD_PAD = 20480
_NBUF = 4


def _leaky(a):
    return jnp.where(a >= 0, a, a * jnp.float32(0.2))


def _pre_body(x_ref, w_ref, a2_ref, h_ref, esd_ref):
    h = jnp.dot(x_ref[...], w_ref[...], preferred_element_type=F32,
                precision=lax.Precision.HIGHEST)
    h_ref[...] = h
    esd_ref[...] = jnp.dot(h, a2_ref[...], preferred_element_type=F32,
                           precision=lax.Precision.HIGHEST)


def _node_mm(x, W, a_src, a_dst, blk=400):
    n, f = x.shape
    h_dim = W.shape[1]
    a2 = jnp.stack([a_src, a_dst], axis=1)
    return pl.pallas_call(
        _pre_body,
        grid=(n // blk,),
        in_specs=[
            pl.BlockSpec((blk, f), lambda i: (i, 0)),
            pl.BlockSpec((f, h_dim), lambda i: (0, 0)),
            pl.BlockSpec((h_dim, 2), lambda i: (0, 0)),
        ],
        out_specs=[
            pl.BlockSpec((blk, h_dim), lambda i: (i, 0)),
            pl.BlockSpec((blk, 2), lambda i: (i, 0)),
        ],
        out_shape=[
            jax.ShapeDtypeStruct((n, h_dim), F32),
            jax.ShapeDtypeStruct((n, 2), F32),
        ],
    )(x, W, a2)


def _combine_block(num_ref, dent_ref, h_ref, esd_ref, b_ref):
    nsum = num_ref[0] + num_ref[1]
    dsum = dent_ref[:, 0:1] + dent_ref[:, 1:2]
    a_self = esd_ref[:, 0:1] + esd_ref[:, 1:2]
    p_self = jnp.exp(_leaky(a_self))
    out = (nsum + p_self * h_ref[...]) / (dsum + p_self)
    return out + b_ref[...]


def _mid_body(num_ref, dent_ref, h_ref, esd_ref, b_ref, w_ref, a2_ref,
              h2_ref, esd2_ref):
    x2 = jnp.tanh(_combine_block(num_ref, dent_ref, h_ref, esd_ref, b_ref))
    h2 = jnp.dot(x2, w_ref[...], preferred_element_type=F32,
                 precision=lax.Precision.HIGHEST)
    h2_ref[...] = h2
    esd2_ref[...] = jnp.dot(h2, a2_ref[...], preferred_element_type=F32,
                            precision=lax.Precision.HIGHEST)


def _mid(num, den_t, h, esd, b, W2, a_src2, a_dst2, blk=400):
    n, h_dim = h.shape
    a2 = jnp.stack([a_src2, a_dst2], axis=1)
    return pl.pallas_call(
        _mid_body,
        grid=(n // blk,),
        in_specs=[
            pl.BlockSpec((2, blk, h_dim), lambda i: (0, i, 0)),
            pl.BlockSpec((blk, 2), lambda i: (i, 0)),
            pl.BlockSpec((blk, h_dim), lambda i: (i, 0)),
            pl.BlockSpec((blk, 2), lambda i: (i, 0)),
            pl.BlockSpec((1, h_dim), lambda i: (0, 0)),
            pl.BlockSpec((h_dim, h_dim), lambda i: (0, 0)),
            pl.BlockSpec((h_dim, 2), lambda i: (0, 0)),
        ],
        out_specs=[
            pl.BlockSpec((blk, h_dim), lambda i: (i, 0)),
            pl.BlockSpec((blk, 2), lambda i: (i, 0)),
        ],
        out_shape=[
            jax.ShapeDtypeStruct((n, h_dim), F32),
            jax.ShapeDtypeStruct((n, 2), F32),
        ],
    )(num, den_t, h, esd, b.reshape(1, h_dim), W2, a2)


def _post_body(num_ref, dent_ref, h_ref, esd_ref, b_ref, wlin_ref, blin_ref,
               out_ref):
    xx = _combine_block(num_ref, dent_ref, h_ref, esd_ref, b_ref)
    out_ref[...] = jnp.dot(xx, wlin_ref[...], preferred_element_type=F32,
                           precision=lax.Precision.HIGHEST) + blin_ref[...]


def _post(num, den_t, h, esd, b, Wlin, blin, blk=400):
    n, h_dim = h.shape
    t = Wlin.shape[1]
    return pl.pallas_call(
        _post_body,
        grid=(n // blk,),
        in_specs=[
            pl.BlockSpec((2, blk, h_dim), lambda i: (0, i, 0)),
            pl.BlockSpec((blk, 2), lambda i: (i, 0)),
            pl.BlockSpec((blk, h_dim), lambda i: (i, 0)),
            pl.BlockSpec((blk, 2), lambda i: (i, 0)),
            pl.BlockSpec((1, h_dim), lambda i: (0, 0)),
            pl.BlockSpec((h_dim, t), lambda i: (0, 0)),
            pl.BlockSpec((1, t), lambda i: (0, 0)),
        ],
        out_specs=pl.BlockSpec((blk, t), lambda i: (i, 0)),
        out_shape=jax.ShapeDtypeStruct((n, t), F32),
    )(num, den_t, h, esd, b.reshape(1, h_dim), Wlin, blin.reshape(1, t))


def _edge_body(h_hbm, esd_hbm, src_hbm, dst_hbm, num_hbm, den_hbm,
               esd_loc, srcl, dstl, pbuf,
               rows0, rows1, rows2, rows3, zbuf,
               num_sh, den_sh,
               sem0, sem1, sem2, sem3):
    core = lax.axis_index("c")
    sub = lax.axis_index("s")
    wid = core * _NSUB + sub
    rows = (rows0, rows1, rows2, rows3)
    sems = (sem0, sem1, sem2, sem3)

    gbase = wid * _GROUPS_PER_TILE
    pltpu.sync_copy(src_hbm.at[pl.ds(gbase, _GROUPS_PER_TILE)], srcl)
    pltpu.sync_copy(dst_hbm.at[pl.ds(gbase, _GROUPS_PER_TILE)], dstl)

    @pl.loop(0, _G)
    def _zero_rows(r):
        rows0[r, pl.ds(0, 16)] = jnp.zeros((16,), F32)
        rows0[r, pl.ds(16, 16)] = jnp.zeros((16,), F32)

    @pl.loop(0, _ROWS_T, step=16)
    def _zero_zbuf(i):
        zbuf[pl.ds(i, 16)] = jnp.zeros((16,), F32)

    base = sub * _ROWS_T
    for i in range(_ROWS_T // _G):
        pltpu.sync_copy(rows0, num_sh.at[pl.ds(base + i * _G, _G)])
    pltpu.sync_copy(zbuf, den_sh.at[pl.ds(base, _ROWS_T)])

    pltpu.sync_copy(esd_hbm, esd_loc)

    plsc.subcore_barrier()

    for g in range(_NBUF - 1):
        pltpu.async_copy(h_hbm.at[srcl.at[g]], rows[g], sems[g])

    def _body(g, b):
        pltpu.make_async_copy(h_hbm.at[srcl.at[g]], rows[b], sems[b]).wait()

        @pl.when(g < _GROUPS_PER_TILE - (_NBUF - 1))
        def _():
            nb = (b + _NBUF - 1) % _NBUF
            pltpu.async_copy(h_hbm.at[srcl.at[g + _NBUF - 1]], rows[nb],
                             sems[nb])

        @pl.loop(0, _G, step=16)
        def _logits(i):
            sv = srcl[g, pl.ds(i, 16)]
            dv = dstl[g, pl.ds(i, 16)]
            ea = plsc.load_gather(esd_loc, [sv * 2])
            eb = plsc.load_gather(esd_loc, [dv * 2 + 1])
            pbuf[0, pl.ds(i, 16)] = jnp.exp(_leaky(ea + eb))

        @pl.loop(0, _G, step=16)
        def _scale(r0):
            pv = pbuf[0, pl.ds(r0, 16)]
            for u in range(16):
                ps = pv[u]
                rows[b][r0 + u, pl.ds(0, 16)] = \
                    rows[b][r0 + u, pl.ds(0, 16)] * ps
                rows[b][r0 + u, pl.ds(16, 16)] = \
                    rows[b][r0 + u, pl.ds(16, 16)] * ps

        pltpu.sync_copy(rows[b], num_sh.at[dstl.at[g]], add=True)
        pltpu.sync_copy(pbuf.at[0], den_sh.at[dstl.at[g]], add=True)

    @pl.loop(0, _GROUPS_PER_TILE, step=_NBUF)
    def _group(g):
        for u in range(_NBUF):
            _body(g + u, u)

    plsc.subcore_barrier()

    hbase = core * _N + base

    @pl.when(sub < _NSUB - 1)
    def _():
        pltpu.sync_copy(num_sh.at[pl.ds(base, _ROWS_T)],
                        num_hbm.at[pl.ds(hbase, _ROWS_T)])
        pltpu.sync_copy(den_sh.at[pl.ds(base, _ROWS_T)],
                        den_hbm.at[pl.ds(hbase, _ROWS_T)])

    @pl.when(sub == _NSUB - 1)
    def _():
        last = _N - (_NSUB - 1) * _ROWS_T
        pltpu.sync_copy(num_sh.at[pl.ds(base, last)],
                        num_hbm.at[pl.ds(hbase, last)])
        pltpu.sync_copy(den_sh.at[pl.ds(base, last)],
                        den_hbm.at[pl.ds(hbase, last)])


def _sc_compiler_params():
    cp = pltpu.CompilerParams()
    fields = pltpu.CompilerParams.__dataclass_fields__
    if "needs_layout_passes" in fields:
        cp = dataclasses.replace(cp, needs_layout_passes=False)
    if "use_tc_tiling_on_sc" in fields:
        cp = dataclasses.replace(cp, use_tc_tiling_on_sc=False)
    return cp


def _edge_pass(h, esd_flat, src2d, dst2d):
    mesh = plsc.VectorSubcoreMesh(core_axis_name="c", subcore_axis_name="s")
    fn = pl.kernel(
        _edge_body,
        mesh=mesh,
        compiler_params=_sc_compiler_params(),
        out_type=[
            jax.ShapeDtypeStruct((_NCORE * _N, _H), F32),
            jax.ShapeDtypeStruct((_NCORE * _N,), F32),
        ],
        scratch_types=[
            pltpu.VMEM((_ESD_PAD,), F32),
            pltpu.VMEM((_GROUPS_PER_TILE, _G), jnp.int32),
            pltpu.VMEM((_GROUPS_PER_TILE, _G), jnp.int32),
            pltpu.VMEM((1, _G), F32),
            pltpu.VMEM((_G, _H), F32),
            pltpu.VMEM((_G, _H), F32),
            pltpu.VMEM((_G, _H), F32),
            pltpu.VMEM((_G, _H), F32),
            pltpu.VMEM((_ROWS_T,), F32),
            pltpu.VMEM_SHARED((_N_SH, _H), F32),
            pltpu.VMEM_SHARED((_N_SH,), F32),
            pltpu.SemaphoreType.DMA,
            pltpu.SemaphoreType.DMA,
            pltpu.SemaphoreType.DMA,
            pltpu.SemaphoreType.DMA,
        ],
    )
    return fn(h, esd_flat, src2d, dst2d)


def kernel(x, edge_index, W1, a_src1, a_dst1, b1, W2, a_src2, a_dst2, b2,
           Wlin, blin):
    src = edge_index[0]
    dst = edge_index[1]
    pad = _E_PAD - _E
    src2d = jnp.concatenate(
        [src, jnp.zeros((pad,), jnp.int32)]).reshape(_E_PAD // _G, _G)
    dst2d = jnp.concatenate(
        [dst, jnp.full((pad,), _N, jnp.int32)]).reshape(_E_PAD // _G, _G)

    h1, esd1 = _node_mm(x, W1, a_src1, a_dst1)
    esd1_flat = jnp.concatenate(
        [esd1.reshape(-1), jnp.zeros((_ESD_PAD - 2 * _N,), F32)])
    num1, den1 = _edge_pass(h1, esd1_flat, src2d, dst2d)
    num1 = num1.reshape(_NCORE, _N, _H)
    den1_t = den1.reshape(_NCORE, _N).T
    h2, esd2 = _mid(num1, den1_t, h1, esd1, b1, W2, a_src2, a_dst2)

    esd2_flat = jnp.concatenate(
        [esd2.reshape(-1), jnp.zeros((_ESD_PAD - 2 * _N,), F32)])
    num2, den2 = _edge_pass(h2, esd2_flat, src2d, dst2d)
    num2 = num2.reshape(_NCORE, _N, _H)
    den2_t = den2.reshape(_NCORE, _N).T
    return _post(num2, den2_t, h2, esd2, b2, Wlin, blin)


# speedup vs baseline: 68.4753x; 1.4150x over previous
import dataclasses
import functools

import jax
import jax.numpy as jnp
from jax import lax
from jax.experimental import pallas as pl
from jax.experimental.pallas import tpu as pltpu
from jax.experimental.pallas import tpu_sc as plsc

F32 = jnp.float32

_N = 10000
_E = 320000
_H = 32
_NCORE = 2
_NSUB = 16
_NTILE = _NCORE * _NSUB
_G = 128
_GROUPS_PER_TILE = 80
_E_PAD = _NTILE * _GROUPS_PER_TILE * _G
_N_SH = 10240
_ROWS_T = _N_SH // _NSUB
_ESD_PAD = 20480
_NBUF = 4


def _leaky(a):
    return jnp.where(a >= 0, a, a * jnp.float32(0.2))


def _pre_body(x_ref, w_ref, a2_ref, h_ref, esd_ref):
    h = jnp.dot(x_ref[...], w_ref[...], preferred_element_type=F32,
                precision=lax.Precision.HIGHEST)
    h_ref[...] = h
    esd_ref[...] = jnp.dot(h, a2_ref[...], preferred_element_type=F32,
                           precision=lax.Precision.HIGHEST)


def _node_mm(x, W, a_src, a_dst, blk=400):
    n, f = x.shape
    h_dim = W.shape[1]
    a2 = jnp.stack([a_src, a_dst], axis=1)
    return pl.pallas_call(
        _pre_body,
        grid=(n // blk,),
        in_specs=[
            pl.BlockSpec((blk, f), lambda i: (i, 0)),
            pl.BlockSpec((f, h_dim), lambda i: (0, 0)),
            pl.BlockSpec((h_dim, 2), lambda i: (0, 0)),
        ],
        out_specs=[
            pl.BlockSpec((blk, h_dim), lambda i: (i, 0)),
            pl.BlockSpec((blk, 2), lambda i: (i, 0)),
        ],
        out_shape=[
            jax.ShapeDtypeStruct((n, h_dim), F32),
            jax.ShapeDtypeStruct((n, 2), F32),
        ],
    )(x, W, a2)


def _combine_block(num_ref, dent_ref, h_ref, esd_ref, b_ref):
    nsum = num_ref[0] + num_ref[1]
    dsum = dent_ref[:, 0:1] + dent_ref[:, 1:2]
    a_self = esd_ref[:, 0:1] + esd_ref[:, 1:2]
    p_self = jnp.exp(_leaky(a_self))
    out = (nsum + p_self * h_ref[...]) / (dsum + p_self)
    return out + b_ref[...]


def _mid_body(num_ref, dent_ref, h_ref, esd_ref, b_ref, w_ref, a2_ref,
              h2_ref, esd2_ref):
    x2 = jnp.tanh(_combine_block(num_ref, dent_ref, h_ref, esd_ref, b_ref))
    h2 = jnp.dot(x2, w_ref[...], preferred_element_type=F32,
                 precision=lax.Precision.HIGHEST)
    h2_ref[...] = h2
    esd2_ref[...] = jnp.dot(h2, a2_ref[...], preferred_element_type=F32,
                            precision=lax.Precision.HIGHEST)


def _mid(num, den_t, h, esd, b, W2, a_src2, a_dst2, blk=400):
    n, h_dim = h.shape
    a2 = jnp.stack([a_src2, a_dst2], axis=1)
    return pl.pallas_call(
        _mid_body,
        grid=(n // blk,),
        in_specs=[
            pl.BlockSpec((2, blk, h_dim), lambda i: (0, i, 0)),
            pl.BlockSpec((blk, 2), lambda i: (i, 0)),
            pl.BlockSpec((blk, h_dim), lambda i: (i, 0)),
            pl.BlockSpec((blk, 2), lambda i: (i, 0)),
            pl.BlockSpec((1, h_dim), lambda i: (0, 0)),
            pl.BlockSpec((h_dim, h_dim), lambda i: (0, 0)),
            pl.BlockSpec((h_dim, 2), lambda i: (0, 0)),
        ],
        out_specs=[
            pl.BlockSpec((blk, h_dim), lambda i: (i, 0)),
            pl.BlockSpec((blk, 2), lambda i: (i, 0)),
        ],
        out_shape=[
            jax.ShapeDtypeStruct((n, h_dim), F32),
            jax.ShapeDtypeStruct((n, 2), F32),
        ],
    )(num, den_t, h, esd, b.reshape(1, h_dim), W2, a2)


def _post_body(num_ref, dent_ref, h_ref, esd_ref, b_ref, wlin_ref, blin_ref,
               out_ref):
    xx = _combine_block(num_ref, dent_ref, h_ref, esd_ref, b_ref)
    out_ref[...] = jnp.dot(xx, wlin_ref[...], preferred_element_type=F32,
                           precision=lax.Precision.HIGHEST) + blin_ref[...]


def _post(num, den_t, h, esd, b, Wlin, blin, blk=400):
    n, h_dim = h.shape
    t = Wlin.shape[1]
    return pl.pallas_call(
        _post_body,
        grid=(n // blk,),
        in_specs=[
            pl.BlockSpec((2, blk, h_dim), lambda i: (0, i, 0)),
            pl.BlockSpec((blk, 2), lambda i: (i, 0)),
            pl.BlockSpec((blk, h_dim), lambda i: (i, 0)),
            pl.BlockSpec((blk, 2), lambda i: (i, 0)),
            pl.BlockSpec((1, h_dim), lambda i: (0, 0)),
            pl.BlockSpec((h_dim, t), lambda i: (0, 0)),
            pl.BlockSpec((1, t), lambda i: (0, 0)),
        ],
        out_specs=pl.BlockSpec((blk, t), lambda i: (i, 0)),
        out_shape=jax.ShapeDtypeStruct((n, t), F32),
    )(num, den_t, h, esd, b.reshape(1, h_dim), Wlin, blin.reshape(1, t))


def _edge_body(h_hbm, esd_hbm, src_hbm, dst_hbm, num_hbm, den_hbm,
               esd_loc, srcl, dstl, pbuf,
               rows0, rows1, rows2, rows3, zbuf,
               num_sh, den_sh, h_sh,
               sem0, sem1, sem2, sem3):
    core = lax.axis_index("c")
    sub = lax.axis_index("s")
    wid = core * _NSUB + sub
    rows = (rows0, rows1, rows2, rows3)
    sems = (sem0, sem1, sem2, sem3)

    gbase = wid * _GROUPS_PER_TILE
    pltpu.sync_copy(src_hbm.at[pl.ds(gbase, _GROUPS_PER_TILE)], srcl)
    pltpu.sync_copy(dst_hbm.at[pl.ds(gbase, _GROUPS_PER_TILE)], dstl)

    @pl.loop(0, _G)
    def _zero_rows(r):
        rows0[r, pl.ds(0, 16)] = jnp.zeros((16,), F32)
        rows0[r, pl.ds(16, 16)] = jnp.zeros((16,), F32)

    @pl.loop(0, _ROWS_T, step=16)
    def _zero_zbuf(i):
        zbuf[pl.ds(i, 16)] = jnp.zeros((16,), F32)

    base = sub * _ROWS_T
    for i in range(_ROWS_T // _G):
        pltpu.sync_copy(rows0, num_sh.at[pl.ds(base + i * _G, _G)])
    pltpu.sync_copy(zbuf, den_sh.at[pl.ds(base, _ROWS_T)])

    pltpu.sync_copy(esd_hbm, esd_loc)

    hrows = _N // _NSUB
    pltpu.sync_copy(h_hbm.at[pl.ds(sub * hrows, hrows)],
                    h_sh.at[pl.ds(sub * hrows, hrows)])

    plsc.subcore_barrier()

    for g in range(_NBUF - 1):
        pltpu.async_copy(h_sh.at[srcl.at[g]], rows[g], sems[g])

    def _body(g, b):
        pltpu.make_async_copy(h_sh.at[srcl.at[g]], rows[b], sems[b]).wait()

        @pl.when(g < _GROUPS_PER_TILE - (_NBUF - 1))
        def _():
            nb = (b + _NBUF - 1) % _NBUF
            pltpu.async_copy(h_sh.at[srcl.at[g + _NBUF - 1]], rows[nb],
                             sems[nb])

        @pl.loop(0, _G, step=16)
        def _logits(i):
            sv = srcl[g, pl.ds(i, 16)]
            dv = dstl[g, pl.ds(i, 16)]
            ea = plsc.load_gather(esd_loc, [sv * 2])
            eb = plsc.load_gather(esd_loc, [dv * 2 + 1])
            pbuf[0, pl.ds(i, 16)] = jnp.exp(_leaky(ea + eb))

        @pl.loop(0, _G, step=16)
        def _scale(r0):
            pv = pbuf[0, pl.ds(r0, 16)]
            for u in range(16):
                ps = pv[u]
                rows[b][r0 + u, pl.ds(0, 16)] = \
                    rows[b][r0 + u, pl.ds(0, 16)] * ps
                rows[b][r0 + u, pl.ds(16, 16)] = \
                    rows[b][r0 + u, pl.ds(16, 16)] * ps

        pltpu.sync_copy(rows[b], num_sh.at[dstl.at[g]], add=True)
        pltpu.sync_copy(pbuf.at[0], den_sh.at[dstl.at[g]], add=True)

    @pl.loop(0, _GROUPS_PER_TILE, step=_NBUF)
    def _group(g):
        for u in range(_NBUF):
            _body(g + u, u)

    plsc.subcore_barrier()

    hbase = core * _N + base

    @pl.when(sub < _NSUB - 1)
    def _():
        pltpu.sync_copy(num_sh.at[pl.ds(base, _ROWS_T)],
                        num_hbm.at[pl.ds(hbase, _ROWS_T)])
        pltpu.sync_copy(den_sh.at[pl.ds(base, _ROWS_T)],
                        den_hbm.at[pl.ds(hbase, _ROWS_T)])

    @pl.when(sub == _NSUB - 1)
    def _():
        last = _N - (_NSUB - 1) * _ROWS_T
        pltpu.sync_copy(num_sh.at[pl.ds(base, last)],
                        num_hbm.at[pl.ds(hbase, last)])
        pltpu.sync_copy(den_sh.at[pl.ds(base, last)],
                        den_hbm.at[pl.ds(hbase, last)])


def _sc_compiler_params():
    cp = pltpu.CompilerParams()
    fields = pltpu.CompilerParams.__dataclass_fields__
    if "needs_layout_passes" in fields:
        cp = dataclasses.replace(cp, needs_layout_passes=False)
    if "use_tc_tiling_on_sc" in fields:
        cp = dataclasses.replace(cp, use_tc_tiling_on_sc=False)
    return cp


def _edge_pass(h, esd_flat, src2d, dst2d):
    mesh = plsc.VectorSubcoreMesh(core_axis_name="c", subcore_axis_name="s")
    fn = pl.kernel(
        _edge_body,
        mesh=mesh,
        compiler_params=_sc_compiler_params(),
        out_type=[
            jax.ShapeDtypeStruct((_NCORE * _N, _H), F32),
            jax.ShapeDtypeStruct((_NCORE * _N,), F32),
        ],
        scratch_types=[
            pltpu.VMEM((_ESD_PAD,), F32),
            pltpu.VMEM((_GROUPS_PER_TILE, _G), jnp.int32),
            pltpu.VMEM((_GROUPS_PER_TILE, _G), jnp.int32),
            pltpu.VMEM((1, _G), F32),
            pltpu.VMEM((_G, _H), F32),
            pltpu.VMEM((_G, _H), F32),
            pltpu.VMEM((_G, _H), F32),
            pltpu.VMEM((_G, _H), F32),
            pltpu.VMEM((_ROWS_T,), F32),
            pltpu.VMEM_SHARED((_N_SH, _H), F32),
            pltpu.VMEM_SHARED((_N_SH,), F32),
            pltpu.VMEM_SHARED((_N, _H), F32),
            pltpu.SemaphoreType.DMA,
            pltpu.SemaphoreType.DMA,
            pltpu.SemaphoreType.DMA,
            pltpu.SemaphoreType.DMA,
        ],
    )
    return fn(h, esd_flat, src2d, dst2d)


def kernel(x, edge_index, W1, a_src1, a_dst1, b1, W2, a_src2, a_dst2, b2,
           Wlin, blin):
    src = edge_index[0]
    dst = edge_index[1]
    pad = _E_PAD - _E
    src2d = jnp.concatenate(
        [src, jnp.zeros((pad,), jnp.int32)]).reshape(_E_PAD // _G, _G)
    dst2d = jnp.concatenate(
        [dst, jnp.full((pad,), _N, jnp.int32)]).reshape(_E_PAD // _G, _G)

    h1, esd1 = _node_mm(x, W1, a_src1, a_dst1)
    esd1_flat = jnp.concatenate(
        [esd1.reshape(-1), jnp.zeros((_ESD_PAD - 2 * _N,), F32)])
    num1, den1 = _edge_pass(h1, esd1_flat, src2d, dst2d)
    num1 = num1.reshape(_NCORE, _N, _H)
    den1_t = den1.reshape(_NCORE, _N).T
    h2, esd2 = _mid(num1, den1_t, h1, esd1, b1, W2, a_src2, a_dst2)

    esd2_flat = jnp.concatenate(
        [esd2.reshape(-1), jnp.zeros((_ESD_PAD - 2 * _N,), F32)])
    num2, den2 = _edge_pass(h2, esd2_flat, src2d, dst2d)
    num2 = num2.reshape(_NCORE, _N, _H)
    den2_t = den2.reshape(_NCORE, _N).T
    return _post(num2, den2_t, h2, esd2, b2, Wlin, blin)


# speedup vs baseline: 68.8225x; 1.0051x over previous
import dataclasses
import functools

import jax
import jax.numpy as jnp
from jax import lax
from jax.experimental import pallas as pl
from jax.experimental.pallas import tpu as pltpu
from jax.experimental.pallas import tpu_sc as plsc

F32 = jnp.float32

_N = 10000
_E = 320000
_H = 32
_NCORE = 2
_NSUB = 16
_NTILE = _NCORE * _NSUB
_G = 128
_GROUPS_PER_TILE = 80
_E_PAD = _NTILE * _GROUPS_PER_TILE * _G
_N_SH = 10240
_ROWS_T = _N_SH // _NSUB
_ESD_PAD = 20480
_NBUF = 4


def _leaky(a):
    return jnp.where(a >= 0, a, a * jnp.float32(0.2))


def _pre_body(x_ref, w_ref, a2_ref, h_ref, esd_ref):
    h = jnp.dot(x_ref[...], w_ref[...], preferred_element_type=F32,
                precision=lax.Precision.HIGHEST)
    h_ref[...] = h
    esd_ref[...] = jnp.dot(h, a2_ref[...], preferred_element_type=F32,
                           precision=lax.Precision.HIGHEST)


def _node_mm(x, W, a_src, a_dst, blk=400):
    n, f = x.shape
    h_dim = W.shape[1]
    a2 = jnp.stack([a_src, a_dst], axis=1)
    return pl.pallas_call(
        _pre_body,
        grid=(n // blk,),
        in_specs=[
            pl.BlockSpec((blk, f), lambda i: (i, 0)),
            pl.BlockSpec((f, h_dim), lambda i: (0, 0)),
            pl.BlockSpec((h_dim, 2), lambda i: (0, 0)),
        ],
        out_specs=[
            pl.BlockSpec((blk, h_dim), lambda i: (i, 0)),
            pl.BlockSpec((blk, 2), lambda i: (i, 0)),
        ],
        out_shape=[
            jax.ShapeDtypeStruct((n, h_dim), F32),
            jax.ShapeDtypeStruct((n, 2), F32),
        ],
    )(x, W, a2)


def _combine_block(num_ref, dent_ref, h_ref, esd_ref, b_ref):
    nsum = num_ref[0] + num_ref[1]
    dsum = dent_ref[:, 0:1] + dent_ref[:, 1:2]
    a_self = esd_ref[:, 0:1] + esd_ref[:, 1:2]
    p_self = jnp.exp(_leaky(a_self))
    out = (nsum + p_self * h_ref[...]) / (dsum + p_self)
    return out + b_ref[...]


def _mid_body(num_ref, dent_ref, h_ref, esd_ref, b_ref, w_ref, a2_ref,
              h2_ref, esd2_ref):
    x2 = jnp.tanh(_combine_block(num_ref, dent_ref, h_ref, esd_ref, b_ref))
    h2 = jnp.dot(x2, w_ref[...], preferred_element_type=F32,
                 precision=lax.Precision.HIGHEST)
    h2_ref[...] = h2
    esd2_ref[...] = jnp.dot(h2, a2_ref[...], preferred_element_type=F32,
                            precision=lax.Precision.HIGHEST)


def _mid(num, den_t, h, esd, b, W2, a_src2, a_dst2, blk=400):
    n, h_dim = h.shape
    a2 = jnp.stack([a_src2, a_dst2], axis=1)
    return pl.pallas_call(
        _mid_body,
        grid=(n // blk,),
        in_specs=[
            pl.BlockSpec((2, blk, h_dim), lambda i: (0, i, 0)),
            pl.BlockSpec((blk, 2), lambda i: (i, 0)),
            pl.BlockSpec((blk, h_dim), lambda i: (i, 0)),
            pl.BlockSpec((blk, 2), lambda i: (i, 0)),
            pl.BlockSpec((1, h_dim), lambda i: (0, 0)),
            pl.BlockSpec((h_dim, h_dim), lambda i: (0, 0)),
            pl.BlockSpec((h_dim, 2), lambda i: (0, 0)),
        ],
        out_specs=[
            pl.BlockSpec((blk, h_dim), lambda i: (i, 0)),
            pl.BlockSpec((blk, 2), lambda i: (i, 0)),
        ],
        out_shape=[
            jax.ShapeDtypeStruct((n, h_dim), F32),
            jax.ShapeDtypeStruct((n, 2), F32),
        ],
    )(num, den_t, h, esd, b.reshape(1, h_dim), W2, a2)


def _post_body(num_ref, dent_ref, h_ref, esd_ref, b_ref, wlin_ref, blin_ref,
               out_ref):
    xx = _combine_block(num_ref, dent_ref, h_ref, esd_ref, b_ref)
    out_ref[...] = jnp.dot(xx, wlin_ref[...], preferred_element_type=F32,
                           precision=lax.Precision.HIGHEST) + blin_ref[...]


def _post(num, den_t, h, esd, b, Wlin, blin, blk=400):
    n, h_dim = h.shape
    t = Wlin.shape[1]
    return pl.pallas_call(
        _post_body,
        grid=(n // blk,),
        in_specs=[
            pl.BlockSpec((2, blk, h_dim), lambda i: (0, i, 0)),
            pl.BlockSpec((blk, 2), lambda i: (i, 0)),
            pl.BlockSpec((blk, h_dim), lambda i: (i, 0)),
            pl.BlockSpec((blk, 2), lambda i: (i, 0)),
            pl.BlockSpec((1, h_dim), lambda i: (0, 0)),
            pl.BlockSpec((h_dim, t), lambda i: (0, 0)),
            pl.BlockSpec((1, t), lambda i: (0, 0)),
        ],
        out_specs=pl.BlockSpec((blk, t), lambda i: (i, 0)),
        out_shape=jax.ShapeDtypeStruct((n, t), F32),
    )(num, den_t, h, esd, b.reshape(1, h_dim), Wlin, blin.reshape(1, t))


def _edge_body(h_hbm, esd_hbm, pk_hbm, num_hbm, den_hbm,
               esd_loc, pkl, srcl, dstl, pbuf,
               rows0, rows1, rows2, rows3, zbuf,
               num_sh, den_sh, h_sh,
               sem0, sem1, sem2, sem3):
    core = lax.axis_index("c")
    sub = lax.axis_index("s")
    wid = core * _NSUB + sub
    rows = (rows0, rows1, rows2, rows3)
    sems = (sem0, sem1, sem2, sem3)

    gbase = wid * _GROUPS_PER_TILE
    pltpu.sync_copy(pk_hbm.at[pl.ds(gbase, _GROUPS_PER_TILE)], pkl)

    @pl.loop(0, _GROUPS_PER_TILE)
    def _unpack(g):
        @pl.loop(0, _G, step=16)
        def _(i):
            v = pkl[g, pl.ds(i, 16)]
            srcl[g, pl.ds(i, 16)] = v & jnp.int32(0xFFFF)
            dstl[g, pl.ds(i, 16)] = lax.shift_right_logical(v, jnp.int32(16))

    @pl.loop(0, _G)
    def _zero_rows(r):
        rows0[r, pl.ds(0, 16)] = jnp.zeros((16,), F32)
        rows0[r, pl.ds(16, 16)] = jnp.zeros((16,), F32)

    @pl.loop(0, _ROWS_T, step=16)
    def _zero_zbuf(i):
        zbuf[pl.ds(i, 16)] = jnp.zeros((16,), F32)

    base = sub * _ROWS_T
    for i in range(_ROWS_T // _G):
        pltpu.sync_copy(rows0, num_sh.at[pl.ds(base + i * _G, _G)])
    pltpu.sync_copy(zbuf, den_sh.at[pl.ds(base, _ROWS_T)])

    pltpu.sync_copy(esd_hbm, esd_loc)

    hrows = _N // _NSUB
    pltpu.sync_copy(h_hbm.at[pl.ds(sub * hrows, hrows)],
                    h_sh.at[pl.ds(sub * hrows, hrows)])

    plsc.subcore_barrier()

    for g in range(_NBUF - 1):
        pltpu.async_copy(h_sh.at[srcl.at[g]], rows[g], sems[g])

    def _body(g, b):
        pltpu.make_async_copy(h_sh.at[srcl.at[g]], rows[b], sems[b]).wait()

        @pl.when(g < _GROUPS_PER_TILE - (_NBUF - 1))
        def _():
            nb = (b + _NBUF - 1) % _NBUF
            pltpu.async_copy(h_sh.at[srcl.at[g + _NBUF - 1]], rows[nb],
                             sems[nb])

        @pl.loop(0, _G, step=16)
        def _logits(i):
            sv = srcl[g, pl.ds(i, 16)]
            dv = dstl[g, pl.ds(i, 16)]
            ea = plsc.load_gather(esd_loc, [sv * 2])
            eb = plsc.load_gather(esd_loc, [dv * 2 + 1])
            pbuf[0, pl.ds(i, 16)] = jnp.exp(_leaky(ea + eb))

        @pl.loop(0, _G, step=16)
        def _scale(r0):
            pv = pbuf[0, pl.ds(r0, 16)]
            for u in range(16):
                ps = pv[u]
                rows[b][r0 + u, pl.ds(0, 16)] = \
                    rows[b][r0 + u, pl.ds(0, 16)] * ps
                rows[b][r0 + u, pl.ds(16, 16)] = \
                    rows[b][r0 + u, pl.ds(16, 16)] * ps

        pltpu.sync_copy(rows[b], num_sh.at[dstl.at[g]], add=True)
        pltpu.sync_copy(pbuf.at[0], den_sh.at[dstl.at[g]], add=True)

    @pl.loop(0, _GROUPS_PER_TILE, step=_NBUF)
    def _group(g):
        for u in range(_NBUF):
            _body(g + u, u)

    plsc.subcore_barrier()

    hbase = core * _N + base

    @pl.when(sub < _NSUB - 1)
    def _():
        pltpu.sync_copy(num_sh.at[pl.ds(base, _ROWS_T)],
                        num_hbm.at[pl.ds(hbase, _ROWS_T)])
        pltpu.sync_copy(den_sh.at[pl.ds(base, _ROWS_T)],
                        den_hbm.at[pl.ds(hbase, _ROWS_T)])

    @pl.when(sub == _NSUB - 1)
    def _():
        last = _N - (_NSUB - 1) * _ROWS_T
        pltpu.sync_copy(num_sh.at[pl.ds(base, last)],
                        num_hbm.at[pl.ds(hbase, last)])
        pltpu.sync_copy(den_sh.at[pl.ds(base, last)],
                        den_hbm.at[pl.ds(hbase, last)])


def _sc_compiler_params():
    cp = pltpu.CompilerParams()
    fields = pltpu.CompilerParams.__dataclass_fields__
    if "needs_layout_passes" in fields:
        cp = dataclasses.replace(cp, needs_layout_passes=False)
    if "use_tc_tiling_on_sc" in fields:
        cp = dataclasses.replace(cp, use_tc_tiling_on_sc=False)
    return cp


def _edge_pass(h, esd_flat, pk2d):
    mesh = plsc.VectorSubcoreMesh(core_axis_name="c", subcore_axis_name="s")
    fn = pl.kernel(
        _edge_body,
        mesh=mesh,
        compiler_params=_sc_compiler_params(),
        out_type=[
            jax.ShapeDtypeStruct((_NCORE * _N, _H), F32),
            jax.ShapeDtypeStruct((_NCORE * _N,), F32),
        ],
        scratch_types=[
            pltpu.VMEM((_ESD_PAD,), F32),
            pltpu.VMEM((_GROUPS_PER_TILE, _G), jnp.int32),
            pltpu.VMEM((_GROUPS_PER_TILE, _G), jnp.int32),
            pltpu.VMEM((_GROUPS_PER_TILE, _G), jnp.int32),
            pltpu.VMEM((1, _G), F32),
            pltpu.VMEM((_G, _H), F32),
            pltpu.VMEM((_G, _H), F32),
            pltpu.VMEM((_G, _H), F32),
            pltpu.VMEM((_G, _H), F32),
            pltpu.VMEM((_ROWS_T,), F32),
            pltpu.VMEM_SHARED((_N_SH, _H), F32),
            pltpu.VMEM_SHARED((_N_SH,), F32),
            pltpu.VMEM_SHARED((_N, _H), F32),
            pltpu.SemaphoreType.DMA,
            pltpu.SemaphoreType.DMA,
            pltpu.SemaphoreType.DMA,
            pltpu.SemaphoreType.DMA,
        ],
    )
    return fn(h, esd_flat, pk2d)


def kernel(x, edge_index, W1, a_src1, a_dst1, b1, W2, a_src2, a_dst2, b2,
           Wlin, blin):
    src = edge_index[0]
    dst = edge_index[1]
    pad = _E_PAD - _E
    packed = jnp.concatenate(
        [src | (dst << 16),
         jnp.full((pad,), _N << 16, jnp.int32)]).reshape(_E_PAD // _G, _G)

    h1, esd1 = _node_mm(x, W1, a_src1, a_dst1)
    esd1_flat = jnp.concatenate(
        [esd1.reshape(-1), jnp.zeros((_ESD_PAD - 2 * _N,), F32)])
    num1, den1 = _edge_pass(h1, esd1_flat, packed)
    num1 = num1.reshape(_NCORE, _N, _H)
    den1_t = den1.reshape(_NCORE, _N).T
    h2, esd2 = _mid(num1, den1_t, h1, esd1, b1, W2, a_src2, a_dst2)

    esd2_flat = jnp.concatenate(
        [esd2.reshape(-1), jnp.zeros((_ESD_PAD - 2 * _N,), F32)])
    num2, den2 = _edge_pass(h2, esd2_flat, packed)
    num2 = num2.reshape(_NCORE, _N, _H)
    den2_t = den2.reshape(_NCORE, _N).T
    return _post(num2, den2_t, h2, esd2, b2, Wlin, blin)
